# baseline (speedup 1.0000x reference)
import jax
import jax.numpy as jnp
from jax import lax
from jax.experimental import pallas as pl
from jax.experimental.pallas import tpu as pltpu
from jax.experimental.pallas import tpu_sc as plsc

N_NODES = 10000
N_EDGES = 320000
D = 128
DE = 16
HF = 64

NC = 2
NS = 16
EPT = N_EDGES // NS
CH = 80
NCH = EPT // CH
NPT = N_NODES // NS
ZB = 125
CW = 8
NBUF = 4


def _make_sc_pass(with_meta: bool):
    out_type = [jax.ShapeDtypeStruct((NC, N_NODES, HF), jnp.float32)]
    if with_meta:
        out_type += [
            jax.ShapeDtypeStruct((NC, N_NODES, DE), jnp.float32),
            jax.ShapeDtypeStruct((NC, N_NODES, CW), jnp.float32),
        ]
    scratch = [
        pltpu.VMEM((NCH, CH), jnp.int32),
        pltpu.VMEM((NCH, CH), jnp.int32),
        pltpu.VMEM((NBUF, CH, HF), jnp.float32),
        pltpu.VMEM((ZB, HF), jnp.float32),
        pltpu.VMEM_SHARED((N_NODES, HF), jnp.float32),
    ] + [pltpu.SemaphoreType.DMA] * (2 * NBUF)
    if with_meta:
        scratch += [
            pltpu.VMEM((CH, DE), jnp.float32),
            pltpu.VMEM((ZB, DE), jnp.float32),
            pltpu.VMEM((128, CW), jnp.float32),
            pltpu.VMEM_SHARED((N_NODES, DE), jnp.float32),
            pltpu.VMEM_SHARED((N_NODES, CW), jnp.float32),
        ]

    def body(x2_hbm, src_hbm, dst_hbm, *rest):
        if with_meta:
            (ea_hbm, onesz_hbm, out_h, out_e, out_c,
             idx_s, idx_d, rows, zbuf, acc_h, *tail) = rest
            gsem, ssem = tail[:NBUF], tail[NBUF:2 * NBUF]
            eabuf, zbuf_e, ones, acc_e, acc_c = tail[2 * NBUF:]
        else:
            (out_h, idx_s, idx_d, rows, zbuf, acc_h, *tail) = rest
            gsem, ssem = tail[:NBUF], tail[NBUF:2 * NBUF]

        cid = lax.axis_index("c")
        sid = lax.axis_index("s")

        pltpu.sync_copy(src_hbm.at[cid, sid], idx_s)
        pltpu.sync_copy(dst_hbm.at[sid], idx_d)

        z = jnp.zeros((16,), jnp.float32)

        def zrow(i, c):
            for k in range(HF // 16):
                zbuf[i, pl.ds(k * 16, 16)] = z
            return c

        lax.fori_loop(0, ZB, zrow, 0)
        for k in range(NPT // ZB):
            pltpu.sync_copy(zbuf, acc_h.at[pl.ds(sid * NPT + k * ZB, ZB)])

        if with_meta:
            def zea(i, c):
                zbuf_e[i, pl.ds(0, 16)] = z
                return c

            lax.fori_loop(0, ZB, zea, 0)
            for k in range(NPT // ZB):
                pltpu.sync_copy(zbuf_e,
                                acc_e.at[pl.ds(sid * NPT + k * ZB, ZB)])
            pltpu.sync_copy(onesz_hbm.at[1], ones)
            for k in range(NPT // ZB):
                pltpu.sync_copy(ones.at[pl.ds(0, ZB)],
                                acc_c.at[pl.ds(sid * NPT + k * ZB, ZB)])
            pltpu.sync_copy(onesz_hbm.at[0], ones)

        plsc.subcore_barrier()

        gix = lambda j: idx_s.at[j]
        dix = lambda j: idx_d.at[j]

        for b in range(NBUF):
            pltpu.async_copy(x2_hbm.at[gix(b)], rows.at[b], gsem[b])

        def grp(g, c):
            for b in range(NBUF):
                j = g * NBUF + b
                pltpu.make_async_copy(
                    x2_hbm.at[gix(0)], rows.at[b], gsem[b]).wait()
                pltpu.async_copy(rows.at[b], acc_h.at[dix(j)], ssem[b],
                                 add=True)
                if with_meta:
                    in_first = j < (NCH // 2)
                    mine_ea = in_first == (cid == 0)

                    @pl.when(mine_ea)
                    def _():
                        pltpu.sync_copy(ea_hbm.at[sid, j], eabuf)
                        pltpu.sync_copy(eabuf, acc_e.at[dix(j)], add=True)

                    @pl.when(jnp.logical_not(mine_ea))
                    def _():
                        pltpu.sync_copy(ones.at[pl.ds(0, CH)],
                                        acc_c.at[dix(j)], add=True)
            for b in range(NBUF):
                j = g * NBUF + b
                pltpu.make_async_copy(
                    rows.at[b], acc_h.at[dix(j)], ssem[b]).wait()
                nxt = j + NBUF

                @pl.when(nxt < NCH)
                def _():
                    pltpu.async_copy(x2_hbm.at[gix(nxt)], rows.at[b], gsem[b])
            return c

        lax.fori_loop(0, NCH // NBUF, grp, 0)
        for b in range(NCH % NBUF):
            j = (NCH // NBUF) * NBUF + b
            pltpu.make_async_copy(
                x2_hbm.at[gix(0)], rows.at[b], gsem[b]).wait()
            pltpu.async_copy(rows.at[b], acc_h.at[dix(j)], ssem[b], add=True)
            if with_meta:
                mine_ea = (j < (NCH // 2)) == (cid == 0)

                @pl.when(mine_ea)
                def _():
                    pltpu.sync_copy(ea_hbm.at[sid, j], eabuf)
                    pltpu.sync_copy(eabuf, acc_e.at[dix(j)], add=True)

                @pl.when(jnp.logical_not(mine_ea))
                def _():
                    pltpu.sync_copy(ones.at[pl.ds(0, CH)],
                                    acc_c.at[dix(j)], add=True)
            pltpu.make_async_copy(
                rows.at[b], acc_h.at[dix(j)], ssem[b]).wait()
        plsc.subcore_barrier()

        sl = pl.ds(sid * NPT, NPT)
        pltpu.sync_copy(acc_h.at[sl], out_h.at[cid, sl])
        if with_meta:
            pltpu.sync_copy(acc_e.at[sl], out_e.at[cid, sl])
            pltpu.sync_copy(acc_c.at[sl], out_c.at[cid, sl])

    mesh = plsc.VectorSubcoreMesh(core_axis_name="c", subcore_axis_name="s")
    return pl.kernel(body, mesh=mesh, out_type=out_type, scratch_types=scratch,
                     compiler_params=pltpu.CompilerParams(
                         use_tc_tiling_on_sc=False))


_sc_pass_meta = _make_sc_pass(True)
_sc_pass = _make_sc_pass(False)

_RB = 1000


def _fuse_body(h, ph, pe0, pe1, pc0, pc1, wst, wsb, bs, wxt, wxb, we, bn, o):
    hv = h[...]
    h_lo, h_hi = hv[:, :HF], hv[:, HF:]
    sh_lo = ph[0] + h_lo
    sh_hi = ph[1] + h_hi
    cnt = pc0[0][:, :1] + pc1[0][:, :1] + 1.0
    dot = lambda a, b: jnp.dot(a, b, preferred_element_type=jnp.float32)
    num = (dot(sh_lo, wxt[...]) + dot(sh_hi, wxb[...])
           + dot(pe0[0] + pe1[0], we[...]))
    self_ = dot(h_lo, wst[...]) + dot(h_hi, wsb[...])
    o[...] = jnp.maximum(self_ + bs[...] + num / cnt + bn[...], 0.0)


def _tc_fuse(h, ph, pe, pc, W_self, b_self, W_neigh, b_neigh):
    grid = (N_NODES // _RB,)
    row = lambda i: (i, 0)
    part0 = lambda i: (0, i, 0)
    part1 = lambda i: (1, i, 0)
    fixed = lambda i: (0, 0)
    return pl.pallas_call(
        _fuse_body,
        grid=grid,
        in_specs=[
            pl.BlockSpec((_RB, D), row),
            pl.BlockSpec((NC, _RB, HF), part0),
            pl.BlockSpec((1, _RB, DE), part0),
            pl.BlockSpec((1, _RB, DE), part1),
            pl.BlockSpec((1, _RB, CW), part0),
            pl.BlockSpec((1, _RB, CW), part1),
            pl.BlockSpec((HF, D), fixed),
            pl.BlockSpec((HF, D), fixed),
            pl.BlockSpec((1, D), fixed),
            pl.BlockSpec((HF, D), fixed),
            pl.BlockSpec((HF, D), fixed),
            pl.BlockSpec((DE, D), fixed),
            pl.BlockSpec((1, D), fixed),
        ],
        out_specs=pl.BlockSpec((_RB, D), row),
        out_shape=jax.ShapeDtypeStruct((N_NODES, D), jnp.float32),
    )(h, ph, pe, pe, pc, pc, W_self[:HF], W_self[HF:], b_self.reshape(1, D),
      W_neigh[:HF], W_neigh[HF:D], W_neigh[D:], b_neigh.reshape(1, D))


def kernel(x, edge_index, edge_attr,
           W_self1, b_self1, W_neigh1, b_neigh1,
           W_self2, b_self2, W_neigh2, b_neigh2):
    eidx = edge_index.astype(jnp.int32)
    src = jnp.stack([2 * eidx[0], 2 * eidx[0] + 1]).reshape(NC, NS, NCH, CH)
    dst = eidx[1].reshape(NS, NCH, CH)
    ea = edge_attr.reshape(NS, NCH, CH, DE)
    onesz = jnp.stack([jnp.ones((128, CW), jnp.float32),
                       jnp.zeros((128, CW), jnp.float32)])

    ph, pe, pc = _sc_pass_meta(x.reshape(NC * N_NODES, HF), src, dst,
                               ea, onesz)
    h1 = _tc_fuse(x, ph, pe, pc, W_self1, b_self1, W_neigh1, b_neigh1)
    outs = _sc_pass(h1.reshape(NC * N_NODES, HF), src, dst)
    ph2 = outs[0] if isinstance(outs, (list, tuple)) else outs
    return _tc_fuse(h1, ph2, pe, pc, W_self2, b_self2, W_neigh2, b_neigh2)


# speedup vs baseline: 1.0091x; 1.0091x over previous
import jax
import jax.numpy as jnp
from jax import lax
from jax.experimental import pallas as pl
from jax.experimental.pallas import tpu as pltpu
from jax.experimental.pallas import tpu_sc as plsc

N_NODES = 10000
N_EDGES = 320000
D = 128
DE = 16
HF = 64

NC = 2
NS = 16
EPT = N_EDGES // NS
CH = 80
NCH = EPT // CH
NPT = N_NODES // NS
ZB = 125
CW = 8
NBUF = 4


def _make_sc_pass(with_meta: bool):
    out_type = [jax.ShapeDtypeStruct((NC, N_NODES, HF), jnp.float32)]
    if with_meta:
        out_type += [
            jax.ShapeDtypeStruct((NC, N_NODES, DE), jnp.float32),
            jax.ShapeDtypeStruct((NC, N_NODES, CW), jnp.float32),
        ]
    scratch = [
        pltpu.VMEM((NCH, CH), jnp.int32),
        pltpu.VMEM((NCH, CH), jnp.int32),
        pltpu.VMEM((NBUF, CH, HF), jnp.float32),
        pltpu.VMEM((ZB, HF), jnp.float32),
        pltpu.VMEM_SHARED((N_NODES, HF), jnp.float32),
    ] + [pltpu.SemaphoreType.DMA] * (2 * NBUF)
    if with_meta:
        scratch += [
            pltpu.VMEM((CH, DE), jnp.float32),
            pltpu.VMEM((ZB, DE), jnp.float32),
            pltpu.VMEM((128, CW), jnp.float32),
            pltpu.VMEM_SHARED((N_NODES, DE), jnp.float32),
            pltpu.VMEM_SHARED((N_NODES, CW), jnp.float32),
        ]

    def body(x2_hbm, src_hbm, dst_hbm, *rest):
        if with_meta:
            (ea_hbm, onesz_hbm, out_h, out_e, out_c,
             idx_s, idx_d, rows, zbuf, acc_h, *tail) = rest
            gsem, ssem = tail[:NBUF], tail[NBUF:2 * NBUF]
            eabuf, zbuf_e, ones, acc_e, acc_c = tail[2 * NBUF:]
        else:
            (out_h, idx_s, idx_d, rows, zbuf, acc_h, *tail) = rest
            gsem, ssem = tail[:NBUF], tail[NBUF:2 * NBUF]

        cid = lax.axis_index("c")
        sid = lax.axis_index("s")

        pltpu.sync_copy(src_hbm.at[sid], idx_s)
        pltpu.sync_copy(dst_hbm.at[sid], idx_d)

        def xform(i, c):
            for k in range(CH // 16):
                v = idx_s[i, pl.ds(k * 16, 16)]
                idx_s[i, pl.ds(k * 16, 16)] = v * 2 + cid
            return c

        lax.fori_loop(0, NCH, xform, 0)

        z = jnp.zeros((16,), jnp.float32)

        def zrow(i, c):
            for k in range(HF // 16):
                zbuf[i, pl.ds(k * 16, 16)] = z
            return c

        lax.fori_loop(0, ZB, zrow, 0)
        for k in range(NPT // ZB):
            pltpu.sync_copy(zbuf, acc_h.at[pl.ds(sid * NPT + k * ZB, ZB)])

        if with_meta:
            def zea(i, c):
                zbuf_e[i, pl.ds(0, 16)] = z
                return c

            lax.fori_loop(0, ZB, zea, 0)
            for k in range(NPT // ZB):
                pltpu.sync_copy(zbuf_e,
                                acc_e.at[pl.ds(sid * NPT + k * ZB, ZB)])
            pltpu.sync_copy(onesz_hbm.at[1], ones)
            for k in range(NPT // ZB):
                pltpu.sync_copy(ones.at[pl.ds(0, ZB)],
                                acc_c.at[pl.ds(sid * NPT + k * ZB, ZB)])
            pltpu.sync_copy(onesz_hbm.at[0], ones)

        plsc.subcore_barrier()

        gix = lambda j: idx_s.at[j]
        dix = lambda j: idx_d.at[j]

        for b in range(NBUF):
            pltpu.async_copy(x2_hbm.at[gix(b)], rows.at[b], gsem[b])

        def grp(g, c):
            for b in range(NBUF):
                j = g * NBUF + b
                pltpu.make_async_copy(
                    x2_hbm.at[gix(0)], rows.at[b], gsem[b]).wait()
                pltpu.async_copy(rows.at[b], acc_h.at[dix(j)], ssem[b],
                                 add=True)
                if with_meta:
                    in_first = j < (NCH // 2)
                    mine_ea = in_first == (cid == 0)

                    @pl.when(mine_ea)
                    def _():
                        pltpu.sync_copy(ea_hbm.at[sid, j], eabuf)
                        pltpu.sync_copy(eabuf, acc_e.at[dix(j)], add=True)

                    @pl.when(jnp.logical_not(mine_ea))
                    def _():
                        pltpu.sync_copy(ones.at[pl.ds(0, CH)],
                                        acc_c.at[dix(j)], add=True)
            for b in range(NBUF):
                j = g * NBUF + b
                pltpu.make_async_copy(
                    rows.at[b], acc_h.at[dix(j)], ssem[b]).wait()
                nxt = j + NBUF

                @pl.when(nxt < NCH)
                def _():
                    pltpu.async_copy(x2_hbm.at[gix(nxt)], rows.at[b], gsem[b])
            return c

        lax.fori_loop(0, NCH // NBUF, grp, 0)
        for b in range(NCH % NBUF):
            j = (NCH // NBUF) * NBUF + b
            pltpu.make_async_copy(
                x2_hbm.at[gix(0)], rows.at[b], gsem[b]).wait()
            pltpu.async_copy(rows.at[b], acc_h.at[dix(j)], ssem[b], add=True)
            if with_meta:
                mine_ea = (j < (NCH // 2)) == (cid == 0)

                @pl.when(mine_ea)
                def _():
                    pltpu.sync_copy(ea_hbm.at[sid, j], eabuf)
                    pltpu.sync_copy(eabuf, acc_e.at[dix(j)], add=True)

                @pl.when(jnp.logical_not(mine_ea))
                def _():
                    pltpu.sync_copy(ones.at[pl.ds(0, CH)],
                                    acc_c.at[dix(j)], add=True)
            pltpu.make_async_copy(
                rows.at[b], acc_h.at[dix(j)], ssem[b]).wait()
        plsc.subcore_barrier()

        sl = pl.ds(sid * NPT, NPT)
        pltpu.sync_copy(acc_h.at[sl], out_h.at[cid, sl])
        if with_meta:
            pltpu.sync_copy(acc_e.at[sl], out_e.at[cid, sl])
            pltpu.sync_copy(acc_c.at[sl], out_c.at[cid, sl])

    mesh = plsc.VectorSubcoreMesh(core_axis_name="c", subcore_axis_name="s")
    return pl.kernel(body, mesh=mesh, out_type=out_type, scratch_types=scratch,
                     compiler_params=pltpu.CompilerParams(
                         use_tc_tiling_on_sc=False))


_sc_pass_meta = _make_sc_pass(True)
_sc_pass = _make_sc_pass(False)

_RB = 1000


def _fuse_body(h, ph, pe0, pe1, pc0, pc1, wst, wsb, bs, wxt, wxb, we, bn, o):
    hv = h[...]
    h_lo, h_hi = hv[:, :HF], hv[:, HF:]
    sh_lo = ph[0] + h_lo
    sh_hi = ph[1] + h_hi
    cnt = pc0[0][:, :1] + pc1[0][:, :1] + 1.0
    dot = lambda a, b: jnp.dot(a, b, preferred_element_type=jnp.float32)
    num = (dot(sh_lo, wxt[...]) + dot(sh_hi, wxb[...])
           + dot(pe0[0] + pe1[0], we[...]))
    self_ = dot(h_lo, wst[...]) + dot(h_hi, wsb[...])
    o[...] = jnp.maximum(self_ + bs[...] + num / cnt + bn[...], 0.0)


def _tc_fuse(h, ph, pe, pc, W_self, b_self, W_neigh, b_neigh):
    grid = (N_NODES // _RB,)
    row = lambda i: (i, 0)
    part0 = lambda i: (0, i, 0)
    part1 = lambda i: (1, i, 0)
    fixed = lambda i: (0, 0)
    return pl.pallas_call(
        _fuse_body,
        grid=grid,
        in_specs=[
            pl.BlockSpec((_RB, D), row),
            pl.BlockSpec((NC, _RB, HF), part0),
            pl.BlockSpec((1, _RB, DE), part0),
            pl.BlockSpec((1, _RB, DE), part1),
            pl.BlockSpec((1, _RB, CW), part0),
            pl.BlockSpec((1, _RB, CW), part1),
            pl.BlockSpec((HF, D), fixed),
            pl.BlockSpec((HF, D), fixed),
            pl.BlockSpec((1, D), fixed),
            pl.BlockSpec((HF, D), fixed),
            pl.BlockSpec((HF, D), fixed),
            pl.BlockSpec((DE, D), fixed),
            pl.BlockSpec((1, D), fixed),
        ],
        out_specs=pl.BlockSpec((_RB, D), row),
        out_shape=jax.ShapeDtypeStruct((N_NODES, D), jnp.float32),
    )(h, ph, pe, pe, pc, pc, W_self[:HF], W_self[HF:], b_self.reshape(1, D),
      W_neigh[:HF], W_neigh[HF:D], W_neigh[D:], b_neigh.reshape(1, D))


def kernel(x, edge_index, edge_attr,
           W_self1, b_self1, W_neigh1, b_neigh1,
           W_self2, b_self2, W_neigh2, b_neigh2):
    eidx = edge_index.astype(jnp.int32)
    src = eidx[0].reshape(NS, NCH, CH)
    dst = eidx[1].reshape(NS, NCH, CH)
    ea = edge_attr.reshape(NS, NCH, CH, DE)
    onesz = jnp.stack([jnp.ones((128, CW), jnp.float32),
                       jnp.zeros((128, CW), jnp.float32)])

    ph, pe, pc = _sc_pass_meta(x.reshape(NC * N_NODES, HF), src, dst,
                               ea, onesz)
    h1 = _tc_fuse(x, ph, pe, pc, W_self1, b_self1, W_neigh1, b_neigh1)
    outs = _sc_pass(h1.reshape(NC * N_NODES, HF), src, dst)
    ph2 = outs[0] if isinstance(outs, (list, tuple)) else outs
    return _tc_fuse(h1, ph2, pe, pc, W_self2, b_self2, W_neigh2, b_neigh2)


# speedup vs baseline: 1.0227x; 1.0135x over previous
import jax
import jax.numpy as jnp
from jax import lax
from jax.experimental import pallas as pl
from jax.experimental.pallas import tpu as pltpu
from jax.experimental.pallas import tpu_sc as plsc

N_NODES = 10000
N_EDGES = 320000
D = 128
DE = 16
HF = 64

NC = 2
NS = 16
EPT = N_EDGES // NS
CH = 125
NCH = EPT // CH
NPT = N_NODES // NS
ZB = 125
CW = 8
NBUF = 2


def _make_sc_pass(with_meta: bool):
    out_type = [jax.ShapeDtypeStruct((NC, N_NODES, HF), jnp.float32)]
    if with_meta:
        out_type += [
            jax.ShapeDtypeStruct((NC, N_NODES, DE), jnp.float32),
            jax.ShapeDtypeStruct((NC, N_NODES, CW), jnp.float32),
        ]
    scratch = [
        pltpu.VMEM((NCH, CH), jnp.int32),
        pltpu.VMEM((NCH, CH), jnp.int32),
        pltpu.VMEM((NBUF, CH, HF), jnp.float32),
        pltpu.VMEM((ZB, HF), jnp.float32),
        pltpu.VMEM_SHARED((N_NODES, HF), jnp.float32),
    ] + [pltpu.SemaphoreType.DMA] * NBUF
    if with_meta:
        scratch += [
            pltpu.VMEM((CH, DE), jnp.float32),
            pltpu.VMEM((ZB, DE), jnp.float32),
            pltpu.VMEM((128, CW), jnp.float32),
            pltpu.VMEM_SHARED((N_NODES, DE), jnp.float32),
            pltpu.VMEM_SHARED((N_NODES, CW), jnp.float32),
        ]

    def body(x2_hbm, src_hbm, dst_hbm, *rest):
        if with_meta:
            (ea_hbm, onesz_hbm, out_h, out_e, out_c,
             idx_s, idx_d, rows, zbuf, acc_h, *tail) = rest
            gsem = tail[:NBUF]
            eabuf, zbuf_e, ones, acc_e, acc_c = tail[NBUF:]
        else:
            (out_h, idx_s, idx_d, rows, zbuf, acc_h, *tail) = rest
            gsem = tail[:NBUF]

        cid = lax.axis_index("c")
        sid = lax.axis_index("s")

        pltpu.sync_copy(src_hbm.at[cid, sid], idx_s)
        pltpu.sync_copy(dst_hbm.at[sid], idx_d)

        z = jnp.zeros((16,), jnp.float32)

        def zrow(i, c):
            for k in range(HF // 16):
                zbuf[i, pl.ds(k * 16, 16)] = z
            return c

        lax.fori_loop(0, ZB, zrow, 0)
        for k in range(NPT // ZB):
            pltpu.sync_copy(zbuf, acc_h.at[pl.ds(sid * NPT + k * ZB, ZB)])

        if with_meta:
            def zea(i, c):
                zbuf_e[i, pl.ds(0, 16)] = z
                return c

            lax.fori_loop(0, ZB, zea, 0)
            for k in range(NPT // ZB):
                pltpu.sync_copy(zbuf_e,
                                acc_e.at[pl.ds(sid * NPT + k * ZB, ZB)])
            pltpu.sync_copy(onesz_hbm.at[1], ones)
            for k in range(NPT // ZB):
                pltpu.sync_copy(ones.at[pl.ds(0, ZB)],
                                acc_c.at[pl.ds(sid * NPT + k * ZB, ZB)])
            pltpu.sync_copy(onesz_hbm.at[0], ones)

        plsc.subcore_barrier()

        gix = lambda j: idx_s.at[j]
        dix = lambda j: idx_d.at[j]

        for b in range(NBUF):
            pltpu.async_copy(x2_hbm.at[gix(b)], rows.at[b], gsem[b])

        def grp(g, c):
            for b in range(NBUF):
                j = g * NBUF + b
                pltpu.make_async_copy(
                    x2_hbm.at[gix(0)], rows.at[b], gsem[b]).wait()
                pltpu.sync_copy(rows.at[b], acc_h.at[dix(j)], add=True)
                nxt = j + NBUF

                @pl.when(nxt < NCH)
                def _():
                    pltpu.async_copy(x2_hbm.at[gix(nxt)], rows.at[b], gsem[b])

                if with_meta:
                    in_first = j < (NCH // 2)
                    mine_ea = in_first == (cid == 0)

                    @pl.when(mine_ea)
                    def _():
                        pltpu.sync_copy(ea_hbm.at[sid, j], eabuf)
                        pltpu.sync_copy(eabuf, acc_e.at[dix(j)], add=True)

                    @pl.when(jnp.logical_not(mine_ea))
                    def _():
                        pltpu.sync_copy(ones.at[pl.ds(0, CH)],
                                        acc_c.at[dix(j)], add=True)
            return c

        lax.fori_loop(0, NCH // NBUF, grp, 0)
        plsc.subcore_barrier()

        sl = pl.ds(sid * NPT, NPT)
        pltpu.sync_copy(acc_h.at[sl], out_h.at[cid, sl])
        if with_meta:
            pltpu.sync_copy(acc_e.at[sl], out_e.at[cid, sl])
            pltpu.sync_copy(acc_c.at[sl], out_c.at[cid, sl])

    mesh = plsc.VectorSubcoreMesh(core_axis_name="c", subcore_axis_name="s")
    return pl.kernel(body, mesh=mesh, out_type=out_type, scratch_types=scratch,
                     compiler_params=pltpu.CompilerParams(
                         use_tc_tiling_on_sc=False))


_sc_pass_meta = _make_sc_pass(True)
_sc_pass = _make_sc_pass(False)

_RB = 1000


def _fuse_body(h, ph, pe0, pe1, pc0, pc1, wst, wsb, bs, wxt, wxb, we, bn, o):
    hv = h[...]
    h_lo, h_hi = hv[:, :HF], hv[:, HF:]
    sh_lo = ph[0] + h_lo
    sh_hi = ph[1] + h_hi
    cnt = pc0[0][:, :1] + pc1[0][:, :1] + 1.0
    dot = lambda a, b: jnp.dot(a, b, preferred_element_type=jnp.float32)
    num = (dot(sh_lo, wxt[...]) + dot(sh_hi, wxb[...])
           + dot(pe0[0] + pe1[0], we[...]))
    self_ = dot(h_lo, wst[...]) + dot(h_hi, wsb[...])
    o[...] = jnp.maximum(self_ + bs[...] + num / cnt + bn[...], 0.0)


def _tc_fuse(h, ph, pe, pc, W_self, b_self, W_neigh, b_neigh):
    grid = (N_NODES // _RB,)
    row = lambda i: (i, 0)
    part0 = lambda i: (0, i, 0)
    part1 = lambda i: (1, i, 0)
    fixed = lambda i: (0, 0)
    return pl.pallas_call(
        _fuse_body,
        grid=grid,
        in_specs=[
            pl.BlockSpec((_RB, D), row),
            pl.BlockSpec((NC, _RB, HF), part0),
            pl.BlockSpec((1, _RB, DE), part0),
            pl.BlockSpec((1, _RB, DE), part1),
            pl.BlockSpec((1, _RB, CW), part0),
            pl.BlockSpec((1, _RB, CW), part1),
            pl.BlockSpec((HF, D), fixed),
            pl.BlockSpec((HF, D), fixed),
            pl.BlockSpec((1, D), fixed),
            pl.BlockSpec((HF, D), fixed),
            pl.BlockSpec((HF, D), fixed),
            pl.BlockSpec((DE, D), fixed),
            pl.BlockSpec((1, D), fixed),
        ],
        out_specs=pl.BlockSpec((_RB, D), row),
        out_shape=jax.ShapeDtypeStruct((N_NODES, D), jnp.float32),
    )(h, ph, pe, pe, pc, pc, W_self[:HF], W_self[HF:], b_self.reshape(1, D),
      W_neigh[:HF], W_neigh[HF:D], W_neigh[D:], b_neigh.reshape(1, D))


def kernel(x, edge_index, edge_attr,
           W_self1, b_self1, W_neigh1, b_neigh1,
           W_self2, b_self2, W_neigh2, b_neigh2):
    eidx = edge_index.astype(jnp.int32)
    src = jnp.stack([2 * eidx[0], 2 * eidx[0] + 1]).reshape(NC, NS, NCH, CH)
    dst = eidx[1].reshape(NS, NCH, CH)
    ea = edge_attr.reshape(NS, NCH, CH, DE)
    onesz = jnp.stack([jnp.ones((128, CW), jnp.float32),
                       jnp.zeros((128, CW), jnp.float32)])

    ph, pe, pc = _sc_pass_meta(x.reshape(NC * N_NODES, HF), src, dst,
                               ea, onesz)
    h1 = _tc_fuse(x, ph, pe, pc, W_self1, b_self1, W_neigh1, b_neigh1)
    outs = _sc_pass(h1.reshape(NC * N_NODES, HF), src, dst)
    ph2 = outs[0] if isinstance(outs, (list, tuple)) else outs
    return _tc_fuse(h1, ph2, pe, pc, W_self2, b_self2, W_neigh2, b_neigh2)
